# baseline (speedup 1.0000x reference)
import functools

import jax
import jax.numpy as jnp
from jax import lax
from jax.experimental import pallas as pl
from jax.experimental.pallas import tpu as pltpu
from jax.experimental.pallas import tpu_sc as plsc

N_SETS = 16
N_POINTS = 4096
N_MARKS = 3
N_CLASSES = 32
H = W = 512
N_TOTAL = N_SETS * N_POINTS

NC = 2
NS = 16
NW = NC * NS
N_PER_W = N_TOTAL // NW
ROUND = 512
NROUND = N_PER_W // ROUND
LANES = 16
GROUPS = ROUND // LANES

_POS_ROWS = 4
_MARK_ROWS = 8
_N_ROWS = _POS_ROWS + N_MARKS * _MARK_ROWS

_SPATIAL_OFF = (0, 1, W, W + 1)
_PLANE = H * W


_NCOMP = 2 + N_MARKS


def _sc_body(pts_hbm, mask_hbm,
             pos_hbm, mm0_hbm, mm1_hbm, mm2_hbm, out_hbm, *scratch):
  pts_v = scratch[0]
  mask_v = scratch[1]
  p = 2
  idx_a = scratch[p:p + 4]; p += 4
  val_a = scratch[p:p + 4]; p += 4
  idx_b = scratch[p:p + 4]; p += 4
  val_b = scratch[p:p + 4]; p += 4
  out_v = scratch[p:p + 4]; p += 4
  sem_a, sem_b = scratch[p], scratch[p + 1]

  wid = lax.axis_index("s") * NC + lax.axis_index("c")
  base = wid * N_PER_W

  pltpu.sync_copy(pts_hbm.at[pl.ds(base * _NCOMP, N_PER_W * _NCOMP)], pts_v)
  pltpu.sync_copy(mask_hbm.at[pl.ds(base, N_PER_W)], mask_v)

  map_refs = (pos_hbm, mm0_hbm, mm1_hbm, mm2_hbm)

  iota5 = jax.lax.iota(jnp.int32, LANES) * _NCOMP

  def comp(off, r):
    return plsc.load_gather(pts_v, [iota5 + (off * _NCOMP + r)])

  def compute_idx(ci, idx):
    def gbody(g, _):
      off = ci * ROUND + g * LANES
      col = g * LANES
      ty = comp(off, 0) * float(H - 1)
      tx = comp(off, 1) * float(W - 1)
      y0 = jnp.minimum(ty.astype(jnp.int32), H - 2)
      x0 = jnp.minimum(tx.astype(jnp.int32), W - 2)
      s00 = y0 * W + x0
      for k in range(_POS_ROWS):
        idx[0][pl.ds(k * ROUND + col, LANES)] = s00 + _SPATIAL_OFF[k]
      for i in range(N_MARKS):
        c = comp(off, 2 + i) * float(N_CLASSES - 1)
        c0 = jnp.minimum(c.astype(jnp.int32), N_CLASSES - 2)
        b = c0 * _PLANE + s00
        for k in range(_MARK_ROWS):
          o = _SPATIAL_OFF[k % 4] + (_PLANE if k >= 4 else 0)
          idx[1 + i][pl.ds(k * ROUND + col, LANES)] = b + o
      return 0

    lax.fori_loop(0, GROUPS, gbody, 0)

  def fire(idx, val, sem):
    for m in range(4):
      pltpu.async_copy(map_refs[m].at[idx[m]], val[m], sem)

  def drain(idx, val, sem):
    for m in range(4):
      pltpu.make_async_copy(map_refs[m].at[idx[m]], val[m], sem).wait()

  def interp(ci, val):
    def gbody(g, _):
      off = ci * ROUND + g * LANES
      col = g * LANES
      ty = comp(off, 0) * float(H - 1)
      tx = comp(off, 1) * float(W - 1)
      y0 = jnp.minimum(ty.astype(jnp.int32), H - 2)
      x0 = jnp.minimum(tx.astype(jnp.int32), W - 2)
      wy = ty - y0.astype(jnp.float32)
      wx = tx - x0.astype(jnp.float32)
      msk = mask_v[pl.ds(off, LANES)]

      def bilerp(v00, v01, v10, v11):
        top = v00 + wx * (v01 - v00)
        bot = v10 + wx * (v11 - v10)
        return top + wy * (bot - top)

      pvals = [val[0][pl.ds(k * ROUND + col, LANES)] for k in range(_POS_ROWS)]
      out_v[0][pl.ds(off, LANES)] = bilerp(*pvals) * msk
      for i in range(N_MARKS):
        c = comp(off, 2 + i) * float(N_CLASSES - 1)
        c0 = jnp.minimum(c.astype(jnp.int32), N_CLASSES - 2)
        wc = c - c0.astype(jnp.float32)
        mvals = [val[1 + i][pl.ds(k * ROUND + col, LANES)]
                 for k in range(_MARK_ROWS)]
        p0 = bilerp(*mvals[0:4])
        p1 = bilerp(*mvals[4:8])
        out_v[1 + i][pl.ds(off, LANES)] = (p0 + wc * (p1 - p0)) * msk
      return 0

    lax.fori_loop(0, GROUPS, gbody, 0)

  compute_idx(0, idx_a)
  fire(idx_a, val_a, sem_a)

  def pair_body(j, _):
    c0 = 2 * j
    compute_idx(c0 + 1, idx_b)
    fire(idx_b, val_b, sem_b)
    drain(idx_a, val_a, sem_a)
    interp(c0, val_a)
    compute_idx(c0 + 2, idx_a)
    fire(idx_a, val_a, sem_a)
    drain(idx_b, val_b, sem_b)
    interp(c0 + 1, val_b)
    return 0

  lax.fori_loop(0, NROUND // 2 - 1, pair_body, 0)

  compute_idx(NROUND - 1, idx_b)
  fire(idx_b, val_b, sem_b)
  drain(idx_a, val_a, sem_a)
  interp(NROUND - 2, val_a)
  drain(idx_b, val_b, sem_b)
  interp(NROUND - 1, val_b)

  for k in range(1 + N_MARKS):
    pltpu.sync_copy(out_v[k], out_hbm.at[pl.ds(k * N_TOTAL + base, N_PER_W)])


@jax.jit
def _sc_call(pts, mask, pos_map, mm0, mm1, mm2):
  mesh = plsc.VectorSubcoreMesh(core_axis_name="c", subcore_axis_name="s")
  return pl.kernel(
      _sc_body,
      out_type=jax.ShapeDtypeStruct(((1 + N_MARKS) * N_TOTAL,), jnp.float32),
      mesh=mesh,
      compiler_params=pltpu.CompilerParams(needs_layout_passes=False),
      scratch_types=(
          [pltpu.VMEM((N_PER_W * _NCOMP,), jnp.float32)]
          + [pltpu.VMEM((N_PER_W,), jnp.float32)]
          + [pltpu.VMEM((_POS_ROWS * ROUND,), jnp.int32)]
          + [pltpu.VMEM((_MARK_ROWS * ROUND,), jnp.int32) for _ in range(3)]
          + [pltpu.VMEM((_POS_ROWS * ROUND,), jnp.float32)]
          + [pltpu.VMEM((_MARK_ROWS * ROUND,), jnp.float32) for _ in range(3)]
          + [pltpu.VMEM((_POS_ROWS * ROUND,), jnp.int32)]
          + [pltpu.VMEM((_MARK_ROWS * ROUND,), jnp.int32) for _ in range(3)]
          + [pltpu.VMEM((_POS_ROWS * ROUND,), jnp.float32)]
          + [pltpu.VMEM((_MARK_ROWS * ROUND,), jnp.float32) for _ in range(3)]
          + [pltpu.VMEM((N_PER_W,), jnp.float32) for _ in range(4)]
          + [pltpu.SemaphoreType.DMA, pltpu.SemaphoreType.DMA]
      ),
  )(pts, mask, pos_map, mm0, mm1, mm2)


def kernel(points, points_mask, position_energy_map,
           marks_energy_map_0, marks_energy_map_1, marks_energy_map_2):
  pts = points.reshape(N_TOTAL * _NCOMP)
  mask = points_mask.reshape(N_TOTAL)
  pos_map = position_energy_map.reshape(H * W)
  mm0 = marks_energy_map_0.reshape(N_CLASSES * H * W)
  mm1 = marks_energy_map_1.reshape(N_CLASSES * H * W)
  mm2 = marks_energy_map_2.reshape(N_CLASSES * H * W)
  out = _sc_call(pts, mask, pos_map, mm0, mm1, mm2)
  return out.reshape(1 + N_MARKS, N_SETS, N_POINTS)


# speedup vs baseline: 1.1134x; 1.1134x over previous
import functools

import jax
import jax.numpy as jnp
from jax import lax
from jax.experimental import pallas as pl
from jax.experimental.pallas import tpu as pltpu
from jax.experimental.pallas import tpu_sc as plsc

N_SETS = 16
N_POINTS = 4096
N_MARKS = 3
N_CLASSES = 32
H = W = 512
N_TOTAL = N_SETS * N_POINTS

NC = 2
NS = 16
NW = NC * NS
N_PER_W = N_TOTAL // NW
ROUND = 512
NROUND = N_PER_W // ROUND
LANES = 16
GROUPS = ROUND // LANES

_POS_ROWS = 4
_MARK_ROWS = 8
_N_ROWS = _POS_ROWS + N_MARKS * _MARK_ROWS

_SPATIAL_OFF = (0, 1, W, W + 1)
_PLANE = H * W


def _sc_body(py_hbm, px_hbm, c0_hbm, c1_hbm, c2_hbm, mask_hbm,
             pos_hbm, mm0_hbm, mm1_hbm, mm2_hbm, out_hbm, *scratch):
  pts_v = scratch[0:5]
  mask_v = scratch[5]
  p = 6
  idx_a = scratch[p:p + 4]; p += 4
  val_a = scratch[p:p + 4]; p += 4
  idx_b = scratch[p:p + 4]; p += 4
  val_b = scratch[p:p + 4]; p += 4
  out_v = scratch[p:p + 4]; p += 4
  sem_a, sem_b = scratch[p], scratch[p + 1]

  wid = lax.axis_index("s") * NC + lax.axis_index("c")
  base = wid * N_PER_W

  comp_hbm = (py_hbm, px_hbm, c0_hbm, c1_hbm, c2_hbm)
  for r in range(5):
    pltpu.sync_copy(comp_hbm[r].at[pl.ds(base, N_PER_W)], pts_v[r])
  pltpu.sync_copy(mask_hbm.at[pl.ds(base, N_PER_W)], mask_v)

  map_refs = (pos_hbm, mm0_hbm, mm1_hbm, mm2_hbm)

  def compute_idx(ci, idx):
    def gbody(g, _):
      off = ci * ROUND + g * LANES
      col = g * LANES
      ty = pts_v[0][pl.ds(off, LANES)] * float(H - 1)
      tx = pts_v[1][pl.ds(off, LANES)] * float(W - 1)
      y0 = jnp.minimum(ty.astype(jnp.int32), H - 2)
      x0 = jnp.minimum(tx.astype(jnp.int32), W - 2)
      s00 = y0 * W + x0
      for k in range(_POS_ROWS):
        idx[0][pl.ds(k * ROUND + col, LANES)] = s00 + _SPATIAL_OFF[k]
      for i in range(N_MARKS):
        c = pts_v[2 + i][pl.ds(off, LANES)] * float(N_CLASSES - 1)
        c0 = jnp.minimum(c.astype(jnp.int32), N_CLASSES - 2)
        b = c0 * _PLANE + s00
        for k in range(_MARK_ROWS):
          o = _SPATIAL_OFF[k % 4] + (_PLANE if k >= 4 else 0)
          idx[1 + i][pl.ds(k * ROUND + col, LANES)] = b + o
      return 0

    lax.fori_loop(0, GROUPS, gbody, 0)

  _SUB = 2

  def _seg(m):
    rows = _POS_ROWS if m == 0 else _MARK_ROWS
    return rows * ROUND // _SUB

  def fire(idx, val, sem):
    for m in range(4):
      n = _seg(m)
      for s2 in range(_SUB):
        pltpu.async_copy(map_refs[m].at[idx[m].at[pl.ds(s2 * n, n)]],
                         val[m].at[pl.ds(s2 * n, n)], sem)

  def drain(idx, val, sem):
    for m in range(4):
      n = _seg(m)
      for s2 in range(_SUB):
        pltpu.make_async_copy(map_refs[m].at[idx[m].at[pl.ds(s2 * n, n)]],
                              val[m].at[pl.ds(s2 * n, n)], sem).wait()

  def interp(ci, val):
    def gbody(g, _):
      off = ci * ROUND + g * LANES
      col = g * LANES
      ty = pts_v[0][pl.ds(off, LANES)] * float(H - 1)
      tx = pts_v[1][pl.ds(off, LANES)] * float(W - 1)
      y0 = jnp.minimum(ty.astype(jnp.int32), H - 2)
      x0 = jnp.minimum(tx.astype(jnp.int32), W - 2)
      wy = ty - y0.astype(jnp.float32)
      wx = tx - x0.astype(jnp.float32)
      msk = mask_v[pl.ds(off, LANES)]

      def bilerp(v00, v01, v10, v11):
        top = v00 + wx * (v01 - v00)
        bot = v10 + wx * (v11 - v10)
        return top + wy * (bot - top)

      pvals = [val[0][pl.ds(k * ROUND + col, LANES)] for k in range(_POS_ROWS)]
      out_v[0][pl.ds(off, LANES)] = bilerp(*pvals) * msk
      for i in range(N_MARKS):
        c = pts_v[2 + i][pl.ds(off, LANES)] * float(N_CLASSES - 1)
        c0 = jnp.minimum(c.astype(jnp.int32), N_CLASSES - 2)
        wc = c - c0.astype(jnp.float32)
        mvals = [val[1 + i][pl.ds(k * ROUND + col, LANES)]
                 for k in range(_MARK_ROWS)]
        p0 = bilerp(*mvals[0:4])
        p1 = bilerp(*mvals[4:8])
        out_v[1 + i][pl.ds(off, LANES)] = (p0 + wc * (p1 - p0)) * msk
      return 0

    lax.fori_loop(0, GROUPS, gbody, 0)

  compute_idx(0, idx_a)
  fire(idx_a, val_a, sem_a)

  def pair_body(j, _):
    c0 = 2 * j
    compute_idx(c0 + 1, idx_b)
    fire(idx_b, val_b, sem_b)
    drain(idx_a, val_a, sem_a)
    interp(c0, val_a)
    compute_idx(c0 + 2, idx_a)
    fire(idx_a, val_a, sem_a)
    drain(idx_b, val_b, sem_b)
    interp(c0 + 1, val_b)
    return 0

  lax.fori_loop(0, NROUND // 2 - 1, pair_body, 0)

  compute_idx(NROUND - 1, idx_b)
  fire(idx_b, val_b, sem_b)
  drain(idx_a, val_a, sem_a)
  interp(NROUND - 2, val_a)
  drain(idx_b, val_b, sem_b)
  interp(NROUND - 1, val_b)

  for k in range(1 + N_MARKS):
    pltpu.sync_copy(out_v[k], out_hbm.at[pl.ds(k * N_TOTAL + base, N_PER_W)])


@jax.jit
def _sc_call(py, px, c0, c1, c2, mask, pos_map, mm0, mm1, mm2):
  mesh = plsc.VectorSubcoreMesh(core_axis_name="c", subcore_axis_name="s")
  return pl.kernel(
      _sc_body,
      out_type=jax.ShapeDtypeStruct(((1 + N_MARKS) * N_TOTAL,), jnp.float32),
      mesh=mesh,
      scratch_types=(
          [pltpu.VMEM((N_PER_W,), jnp.float32) for _ in range(5)]
          + [pltpu.VMEM((N_PER_W,), jnp.float32)]
          + [pltpu.VMEM((_POS_ROWS * ROUND,), jnp.int32)]
          + [pltpu.VMEM((_MARK_ROWS * ROUND,), jnp.int32) for _ in range(3)]
          + [pltpu.VMEM((_POS_ROWS * ROUND,), jnp.float32)]
          + [pltpu.VMEM((_MARK_ROWS * ROUND,), jnp.float32) for _ in range(3)]
          + [pltpu.VMEM((_POS_ROWS * ROUND,), jnp.int32)]
          + [pltpu.VMEM((_MARK_ROWS * ROUND,), jnp.int32) for _ in range(3)]
          + [pltpu.VMEM((_POS_ROWS * ROUND,), jnp.float32)]
          + [pltpu.VMEM((_MARK_ROWS * ROUND,), jnp.float32) for _ in range(3)]
          + [pltpu.VMEM((N_PER_W,), jnp.float32) for _ in range(4)]
          + [pltpu.SemaphoreType.DMA, pltpu.SemaphoreType.DMA]
      ),
  )(py, px, c0, c1, c2, mask, pos_map, mm0, mm1, mm2)


def kernel(points, points_mask, position_energy_map,
           marks_energy_map_0, marks_energy_map_1, marks_energy_map_2):
  pts = points.reshape(N_TOTAL, 2 + N_MARKS)
  comps = [pts[:, r] for r in range(5)]
  mask = points_mask.reshape(N_TOTAL)
  pos_map = position_energy_map.reshape(H * W)
  mm0 = marks_energy_map_0.reshape(N_CLASSES * H * W)
  mm1 = marks_energy_map_1.reshape(N_CLASSES * H * W)
  mm2 = marks_energy_map_2.reshape(N_CLASSES * H * W)
  out = _sc_call(*comps, mask, pos_map, mm0, mm1, mm2)
  return out.reshape(1 + N_MARKS, N_SETS, N_POINTS)
